# baseline (speedup 1.0000x reference)
import jax
import jax.numpy as jnp
from jax import lax
from jax.experimental import pallas as pl
from jax.experimental.pallas import tpu as pltpu
from jax.experimental.pallas import tpu_sc as plsc

NUM_CORES = 2
NUM_SUBCORES = 16
NUM_WORKERS = NUM_CORES * NUM_SUBCORES

B = 1024
T = 200
D = 128
LANES = 16
CHUNK = 128
TOTAL = B * T
ROWS_PER_W = TOTAL // NUM_WORKERS
CHUNKS_PER_W = ROWS_PER_W // CHUNK
RING = 5
GROUPS = CHUNKS_PER_W // RING


def _embed_kernel(idx_hbm, table_hbm, te_hbm, out_hbm,
                  idx_v, bufs, te_v, table_sh, gsem, osem):
    wid = lax.axis_index("s") * NUM_CORES + lax.axis_index("c")
    row0 = wid * ROWS_PER_W

    @pl.when(lax.axis_index("s") == 1)
    def _():
        pltpu.sync_copy(table_hbm, table_sh)
    pltpu.sync_copy(te_hbm, te_v)
    plsc.subcore_barrier()

    pltpu.sync_copy(idx_hbm.at[wid], idx_v)

    def gather_copy(c, b):
        return pltpu.make_async_copy(
            table_sh.at[idx_v.at[c]], bufs.at[b], gsem.at[b])

    def out_copy(c, b):
        return pltpu.make_async_copy(
            bufs.at[b], out_hbm.at[pl.ds(row0 + c * CHUNK, CHUNK)],
            osem.at[b])

    def add_te(c, b):
        phase = lax.rem(c * CHUNK, T)

        def row(i, r):
            for j in range(D // LANES):
                sl = pl.ds(j * LANES, LANES)
                bufs[b, i, sl] += te_v[r, sl]
            r = r + 1
            return lax.select(r == T, 0, r)

        lax.fori_loop(0, CHUNK, row, phase)

    gather_copy(0, 0).start()
    gather_copy(1, 1).start()

    def group(g, carry):
        for u in range(RING):
            c = g * RING + u
            ob = (u + RING - 2) % RING
            if u >= 2:
                out_copy(c - 2, ob).wait()
            else:
                @pl.when(g >= 1)
                def _():
                    out_copy(c - 2, ob).wait()
            nb = (u + 2) % RING
            if u < RING - 2:
                gather_copy(c + 2, nb).start()
            else:
                @pl.when(g < GROUPS - 1)
                def _():
                    gather_copy(c + 2, nb).start()
            gather_copy(c, u).wait()
            add_te(c, u)
            out_copy(c, u).start()
        return carry

    lax.fori_loop(0, GROUPS, group, 0)

    out_copy(CHUNKS_PER_W - 2, (CHUNKS_PER_W - 2) % RING).wait()
    out_copy(CHUNKS_PER_W - 1, (CHUNKS_PER_W - 1) % RING).wait()


def kernel(actions, embedding_table, time_embed):
    idx = actions.reshape(NUM_WORKERS, CHUNKS_PER_W, CHUNK).astype(jnp.int32)
    te = time_embed.reshape(T, D)

    mesh = plsc.VectorSubcoreMesh(
        core_axis_name="c", subcore_axis_name="s",
        num_cores=NUM_CORES, num_subcores=NUM_SUBCORES,
    )
    out = pl.kernel(
        _embed_kernel,
        out_type=jax.ShapeDtypeStruct((TOTAL, D), jnp.float32),
        mesh=mesh,
        scratch_types=[
            pltpu.VMEM((CHUNKS_PER_W, CHUNK), jnp.int32),
            pltpu.VMEM((RING, CHUNK, D), jnp.float32),
            pltpu.VMEM((T, D), jnp.float32),
            pltpu.VMEM_SHARED((1000, D), jnp.float32),
            pltpu.SemaphoreType.DMA((RING,)),
            pltpu.SemaphoreType.DMA((RING,)),
        ],
    )(idx, embedding_table, te)
    return out.reshape(B, T, D)


# speedup vs baseline: 2.9491x; 2.9491x over previous
import jax
import jax.numpy as jnp
from jax import lax
from jax.experimental import pallas as pl
from jax.experimental.pallas import tpu as pltpu
from jax.experimental.pallas import tpu_sc as plsc

NUM_CORES = 2
NUM_SUBCORES = 16
NUM_WORKERS = NUM_CORES * NUM_SUBCORES

B = 1024
T = 200
D = 128
LANES = 16
CHUNK = 128
TOTAL = B * T
ROWS_PER_W = TOTAL // NUM_WORKERS
CHUNKS_PER_W = ROWS_PER_W // CHUNK
RING = 5
GROUPS = CHUNKS_PER_W // RING


def _embed_kernel(idx_hbm, table_hbm, te_hbm, out_hbm,
                  idx_v, bufs, te_v, table_sh, gsem, osem):
    wid = lax.axis_index("s") * NUM_CORES + lax.axis_index("c")
    row0 = wid * ROWS_PER_W

    @pl.when(lax.axis_index("s") == 1)
    def _():
        pltpu.sync_copy(table_hbm, table_sh)
    pltpu.sync_copy(te_hbm, te_v)
    plsc.subcore_barrier()

    pltpu.sync_copy(idx_hbm.at[wid], idx_v)

    def gather_copy(c, b):
        return pltpu.make_async_copy(
            table_sh.at[idx_v.at[c]], bufs.at[b], gsem.at[b])

    def out_copy(c, b):
        return pltpu.make_async_copy(
            bufs.at[b], out_hbm.at[pl.ds(row0 + c * CHUNK, CHUNK)],
            osem.at[b])

    def add_te(c, b):
        phase = lax.rem(c * CHUNK, T)

        @plsc.parallel_loop(0, CHUNK, unroll=4)
        def _(i):
            s = phase + i
            r = s - lax.select(s >= T, T, 0)
            for j in range(D // LANES):
                sl = pl.ds(j * LANES, LANES)
                bufs[b, i, sl] += te_v[r, sl]

    gather_copy(0, 0).start()
    gather_copy(1, 1).start()

    def group(g, carry):
        for u in range(RING):
            c = g * RING + u
            ob = (u + RING - 2) % RING
            if u >= 2:
                out_copy(c - 2, ob).wait()
            else:
                @pl.when(g >= 1)
                def _():
                    out_copy(c - 2, ob).wait()
            nb = (u + 2) % RING
            if u < RING - 2:
                gather_copy(c + 2, nb).start()
            else:
                @pl.when(g < GROUPS - 1)
                def _():
                    gather_copy(c + 2, nb).start()
            gather_copy(c, u).wait()
            add_te(c, u)
            out_copy(c, u).start()
        return carry

    lax.fori_loop(0, GROUPS, group, 0)

    out_copy(CHUNKS_PER_W - 2, (CHUNKS_PER_W - 2) % RING).wait()
    out_copy(CHUNKS_PER_W - 1, (CHUNKS_PER_W - 1) % RING).wait()


def kernel(actions, embedding_table, time_embed):
    idx = actions.reshape(NUM_WORKERS, CHUNKS_PER_W, CHUNK).astype(jnp.int32)
    te = time_embed.reshape(T, D)

    mesh = plsc.VectorSubcoreMesh(
        core_axis_name="c", subcore_axis_name="s",
        num_cores=NUM_CORES, num_subcores=NUM_SUBCORES,
    )
    out = pl.kernel(
        _embed_kernel,
        out_type=jax.ShapeDtypeStruct((TOTAL, D), jnp.float32),
        mesh=mesh,
        scratch_types=[
            pltpu.VMEM((CHUNKS_PER_W, CHUNK), jnp.int32),
            pltpu.VMEM((RING, CHUNK, D), jnp.float32),
            pltpu.VMEM((T, D), jnp.float32),
            pltpu.VMEM_SHARED((1000, D), jnp.float32),
            pltpu.SemaphoreType.DMA((RING,)),
            pltpu.SemaphoreType.DMA((RING,)),
        ],
    )(idx, embedding_table, te)
    return out.reshape(B, T, D)
